# baseline (speedup 1.0000x reference)
import functools

import jax
import jax.numpy as jnp
from jax.experimental import pallas as pl
from jax.experimental.pallas import tpu as pltpu

HIDDEN = 1024
HEADS = 16
DH = 64
SEG = 128
SCALE = DH ** -0.5
NKEY = HEADS * SEG
SEGS_PER_BLK = 2
BLK = SEG * SEGS_PER_BLK
GHEADS = 4
GROUPS = HEADS // GHEADS
GROWS = GHEADS * SEG
GVAL = GHEADS * DH
GAUG = GVAL + DH


def _fused_kernel(x_ref, wqkvT_ref, woutT_ref, perm_ref, dmap_ref, out_ref,
                  vaug_ref):
    b = pl.program_id(0)
    s = pl.program_id(1)

    @pl.when((b == 0) & (s == 0))
    def _init():
        rr = jax.lax.broadcasted_iota(jnp.int32, (GROWS, GAUG), 0)
        cc = jax.lax.broadcasted_iota(jnp.int32, (GROWS, GAUG), 1)
        ini = (cc == GVAL + rr // SEG).astype(jnp.bfloat16)
        for g in range(SEGS_PER_BLK):
            for gr in range(GROUPS):
                vaug_ref[g, gr] = ini

    xb = x_ref[0].astype(jnp.bfloat16)
    qkv = jnp.dot(xb, wqkvT_ref[...], preferred_element_type=jnp.float32)
    q = qkv[:, :HIDDEN].astype(jnp.bfloat16)
    k = qkv[:, HIDDEN:2 * HIDDEN].astype(jnp.bfloat16)
    v = qkv[:, 2 * HIDDEN:].astype(jnp.bfloat16)

    attn_parts = []
    for g in range(SEGS_PER_BLK):
        r0 = g * SEG
        q_flip = jnp.dot(perm_ref[...], q[r0 + 64:r0 + SEG, :],
                         preferred_element_type=jnp.float32
                         ).astype(jnp.bfloat16)
        q2 = jnp.concatenate([q[r0:r0 + 64, :], q_flip], axis=0)

        parts = []
        for h in range(HEADS):
            sl = slice(h * DH, (h + 1) * DH)
            parts.append(jax.lax.dot_general(
                q2[:, sl], k[r0:r0 + SEG, sl], (((1,), (1,)), ((), ())),
                preferred_element_type=jnp.float32))
            vaug_ref[g, h // GHEADS,
                     (h % GHEADS) * SEG:(h % GHEADS + 1) * SEG,
                     (h % GHEADS) * DH:(h % GHEADS + 1) * DH] = v[r0:r0 + SEG, sl]
        S = jnp.concatenate(parts, axis=1)

        c0 = jnp.max(S[:64, :], axis=0, keepdims=True)
        c1 = jnp.max(S[64:, :], axis=0, keepdims=True)
        W = jnp.exp(S - jnp.concatenate([jnp.broadcast_to(c0, (64, NKEY)),
                                         jnp.broadcast_to(c1, (64, NKEY))],
                                        axis=0))
        Wb = W.astype(jnp.bfloat16)

        for gr in range(GROUPS):
            nd = jnp.dot(Wb[:, gr * GROWS:(gr + 1) * GROWS], vaug_ref[g, gr],
                         preferred_element_type=jnp.float32)
            rec = (1.0 / (1e-10 + nd[:, GVAL:])).astype(jnp.bfloat16)
            rec_bc = jnp.dot(rec, dmap_ref[...],
                             preferred_element_type=jnp.float32)
            attn_parts.append((nd[:, :GVAL] * rec_bc).astype(jnp.bfloat16))

    rows = []
    for g in range(SEGS_PER_BLK):
        rows.append(jnp.concatenate(attn_parts[g * GROUPS:(g + 1) * GROUPS],
                                    axis=1))
    attn = jnp.concatenate(rows, axis=0)
    out_ref[0] = jnp.dot(attn, woutT_ref[...],
                         preferred_element_type=jnp.float32)


@functools.partial(jax.jit, static_argnums=())
def kernel(x, Wqkv, Wout):
    B, M, D = x.shape
    nblk = M // BLK
    scale_vec = jnp.where(jnp.arange(3 * HIDDEN) < HIDDEN, SCALE, 1.0)
    wqkvT = (Wqkv * scale_vec[:, None]).T.astype(jnp.bfloat16)
    woutT = Wout.T.astype(jnp.bfloat16)
    i64 = jnp.arange(64)
    perm = (i64[:, None] + i64[None, :] == 63).astype(jnp.bfloat16)
    dmap = (i64[:, None] == jnp.arange(GVAL)[None, :] // DH
            ).astype(jnp.bfloat16)

    return pl.pallas_call(
        _fused_kernel,
        grid=(B, nblk),
        in_specs=[
            pl.BlockSpec((1, BLK, HIDDEN), lambda b, s: (b, s, 0)),
            pl.BlockSpec((HIDDEN, 3 * HIDDEN), lambda b, s: (0, 0)),
            pl.BlockSpec((HIDDEN, HIDDEN), lambda b, s: (0, 0)),
            pl.BlockSpec((64, 64), lambda b, s: (0, 0)),
            pl.BlockSpec((64, GVAL), lambda b, s: (0, 0)),
        ],
        out_specs=pl.BlockSpec((1, BLK, HIDDEN), lambda b, s: (b, s, 0)),
        out_shape=jax.ShapeDtypeStruct((B, M, D), jnp.float32),
        scratch_shapes=[pltpu.VMEM((SEGS_PER_BLK, GROUPS, GROWS, GAUG),
                                   jnp.bfloat16)],
        compiler_params=pltpu.CompilerParams(
            dimension_semantics=("arbitrary", "arbitrary")),
    )(x, wqkvT, woutT, perm, dmap)


# speedup vs baseline: 1.0683x; 1.0683x over previous
import functools

import jax
import jax.numpy as jnp
from jax.experimental import pallas as pl
from jax.experimental.pallas import tpu as pltpu

HIDDEN = 1024
HEADS = 16
DH = 64
SEG = 128
SCALE = DH ** -0.5
NKEY = HEADS * SEG
VAUG = HIDDEN + SEG
SEGS_PER_BLK = 2
BLK = SEG * SEGS_PER_BLK

_TN = (((1,), (1,)), ((), ()))


def _fused_kernel(x_ref, wqkv_ref, wout_ref, perm_ref, dmap_ref, out_ref,
                  vaug_ref):
    b = pl.program_id(0)
    s = pl.program_id(1)

    @pl.when((b == 0) & (s == 0))
    def _init():
        rr = jax.lax.broadcasted_iota(jnp.int32, (NKEY, VAUG), 0)
        cc = jax.lax.broadcasted_iota(jnp.int32, (NKEY, VAUG), 1)
        ini = (cc == HIDDEN + rr // SEG).astype(jnp.bfloat16)
        for g in range(SEGS_PER_BLK):
            vaug_ref[g] = ini

    xb = x_ref[0].astype(jnp.bfloat16)
    qkv = jax.lax.dot_general(xb, wqkv_ref[...], _TN,
                              preferred_element_type=jnp.float32)
    q = qkv[:, :HIDDEN].astype(jnp.bfloat16)
    k = qkv[:, HIDDEN:2 * HIDDEN].astype(jnp.bfloat16)
    v = qkv[:, 2 * HIDDEN:].astype(jnp.bfloat16)

    attn_parts = []
    for g in range(SEGS_PER_BLK):
        r0 = g * SEG
        q_flip = jnp.dot(perm_ref[...], q[r0 + 64:r0 + SEG, :],
                         preferred_element_type=jnp.float32
                         ).astype(jnp.bfloat16)
        q2 = jnp.concatenate([q[r0:r0 + 64, :], q_flip], axis=0)

        parts = []
        for h in range(HEADS):
            sl = slice(h * DH, (h + 1) * DH)
            parts.append(jax.lax.dot_general(
                q2[:, sl], k[r0:r0 + SEG, sl], _TN,
                preferred_element_type=jnp.float32))
            vaug_ref[g, h * SEG:(h + 1) * SEG, sl] = v[r0:r0 + SEG, sl]
        S = jnp.concatenate(parts, axis=1)

        c0 = jnp.max(S[:64, :], axis=0, keepdims=True)
        c1 = jnp.max(S[64:, :], axis=0, keepdims=True)
        W = jnp.exp(S - jnp.concatenate([jnp.broadcast_to(c0, (64, NKEY)),
                                         jnp.broadcast_to(c1, (64, NKEY))],
                                        axis=0))
        Wb = W.astype(jnp.bfloat16)

        nd = jnp.dot(Wb, vaug_ref[g],
                     preferred_element_type=jnp.float32)
        rec = (1.0 / (1e-10 + nd[:, HIDDEN:HIDDEN + DH])
               ).astype(jnp.bfloat16)
        rec_bc = jnp.dot(rec, dmap_ref[...],
                         preferred_element_type=jnp.float32)
        attn_parts.append((nd[:, :HIDDEN] * rec_bc).astype(jnp.bfloat16))

    attn = jnp.concatenate(attn_parts, axis=0)
    out_ref[0] = jax.lax.dot_general(attn, wout_ref[...], _TN,
                                     preferred_element_type=jnp.float32)


@functools.partial(jax.jit, static_argnums=())
def kernel(x, Wqkv, Wout):
    B, M, D = x.shape
    nblk = M // BLK
    scale_col = jnp.where(jnp.arange(3 * HIDDEN) < HIDDEN, SCALE, 1.0)
    wqkv_bf = (Wqkv * scale_col[:, None]).astype(jnp.bfloat16)
    wout_bf = Wout.astype(jnp.bfloat16)
    i64 = jnp.arange(64)
    perm = (i64[:, None] + i64[None, :] == 63).astype(jnp.bfloat16)
    dmap = (i64[:, None] == jnp.arange(HIDDEN)[None, :] // DH
            ).astype(jnp.bfloat16)

    return pl.pallas_call(
        _fused_kernel,
        grid=(B, nblk),
        in_specs=[
            pl.BlockSpec((1, BLK, HIDDEN), lambda b, s: (b, s, 0)),
            pl.BlockSpec((3 * HIDDEN, HIDDEN), lambda b, s: (0, 0)),
            pl.BlockSpec((HIDDEN, HIDDEN), lambda b, s: (0, 0)),
            pl.BlockSpec((64, 64), lambda b, s: (0, 0)),
            pl.BlockSpec((64, HIDDEN), lambda b, s: (0, 0)),
        ],
        out_specs=pl.BlockSpec((1, BLK, HIDDEN), lambda b, s: (b, s, 0)),
        out_shape=jax.ShapeDtypeStruct((B, M, D), jnp.float32),
        scratch_shapes=[pltpu.VMEM((SEGS_PER_BLK, NKEY, VAUG), jnp.bfloat16)],
        compiler_params=pltpu.CompilerParams(
            dimension_semantics=("arbitrary", "arbitrary")),
    )(x, wqkv_bf, wout_bf, perm, dmap)


# speedup vs baseline: 1.1131x; 1.0420x over previous
import functools

import jax
import jax.numpy as jnp
from jax.experimental import pallas as pl
from jax.experimental.pallas import tpu as pltpu

HIDDEN = 1024
HEADS = 16
DH = 64
SEG = 128
SCALE = DH ** -0.5
NKEY = HEADS * SEG
VAUG = HIDDEN + SEG
SEGS_PER_BLK = 4
BLK = SEG * SEGS_PER_BLK

_TN = (((1,), (1,)), ((), ()))


def _fused_kernel(x_ref, wqkv_ref, wout_ref, perm_ref, dmap_ref, out_ref,
                  vaug_ref):
    b = pl.program_id(0)
    s = pl.program_id(1)

    @pl.when((b == 0) & (s == 0))
    def _init():
        rr = jax.lax.broadcasted_iota(jnp.int32, (NKEY, VAUG), 0)
        cc = jax.lax.broadcasted_iota(jnp.int32, (NKEY, VAUG), 1)
        ini = (cc == HIDDEN + rr // SEG).astype(jnp.bfloat16)
        for g in range(SEGS_PER_BLK):
            vaug_ref[g] = ini

    xb = x_ref[0].astype(jnp.bfloat16)
    qkv = jax.lax.dot_general(xb, wqkv_ref[...], _TN,
                              preferred_element_type=jnp.float32)
    q = qkv[:, :HIDDEN].astype(jnp.bfloat16)
    k = qkv[:, HIDDEN:2 * HIDDEN].astype(jnp.bfloat16)
    v = qkv[:, 2 * HIDDEN:].astype(jnp.bfloat16)

    attn_parts = []
    for g in range(SEGS_PER_BLK):
        r0 = g * SEG
        q_flip = jnp.dot(perm_ref[...], q[r0 + 64:r0 + SEG, :],
                         preferred_element_type=jnp.float32
                         ).astype(jnp.bfloat16)
        q2 = jnp.concatenate([q[r0:r0 + 64, :], q_flip], axis=0)

        parts = []
        for h in range(HEADS):
            sl = slice(h * DH, (h + 1) * DH)
            parts.append(jax.lax.dot_general(
                q2[:, sl], k[r0:r0 + SEG, sl], _TN,
                preferred_element_type=jnp.float32))
            vaug_ref[g, h * SEG:(h + 1) * SEG, sl] = v[r0:r0 + SEG, sl]
        S = jnp.concatenate(parts, axis=1)

        c0 = jnp.max(S[:64, :], axis=0, keepdims=True)
        c1 = jnp.max(S[64:, :], axis=0, keepdims=True)
        W = jnp.exp(S - jnp.concatenate([jnp.broadcast_to(c0, (64, NKEY)),
                                         jnp.broadcast_to(c1, (64, NKEY))],
                                        axis=0))
        Wb = W.astype(jnp.bfloat16)

        nd = jnp.dot(Wb, vaug_ref[g],
                     preferred_element_type=jnp.float32)
        rec = (1.0 / (1e-10 + nd[:, HIDDEN:HIDDEN + DH])
               ).astype(jnp.bfloat16)
        rec_bc = jnp.dot(rec, dmap_ref[...],
                         preferred_element_type=jnp.float32)
        attn_parts.append((nd[:, :HIDDEN] * rec_bc).astype(jnp.bfloat16))

    attn = jnp.concatenate(attn_parts, axis=0)
    out_ref[0] = jax.lax.dot_general(attn, wout_ref[...], _TN,
                                     preferred_element_type=jnp.float32)


@functools.partial(jax.jit, static_argnums=())
def kernel(x, Wqkv, Wout):
    B, M, D = x.shape
    nblk = M // BLK
    scale_col = jnp.where(jnp.arange(3 * HIDDEN) < HIDDEN, SCALE, 1.0)
    wqkv_bf = (Wqkv * scale_col[:, None]).astype(jnp.bfloat16)
    wout_bf = Wout.astype(jnp.bfloat16)
    i64 = jnp.arange(64)
    perm = (i64[:, None] + i64[None, :] == 63).astype(jnp.bfloat16)
    dmap = (i64[:, None] == jnp.arange(HIDDEN)[None, :] // DH
            ).astype(jnp.bfloat16)

    return pl.pallas_call(
        _fused_kernel,
        grid=(B, nblk),
        in_specs=[
            pl.BlockSpec((1, BLK, HIDDEN), lambda b, s: (b, s, 0)),
            pl.BlockSpec((3 * HIDDEN, HIDDEN), lambda b, s: (0, 0)),
            pl.BlockSpec((HIDDEN, HIDDEN), lambda b, s: (0, 0)),
            pl.BlockSpec((64, 64), lambda b, s: (0, 0)),
            pl.BlockSpec((64, HIDDEN), lambda b, s: (0, 0)),
        ],
        out_specs=pl.BlockSpec((1, BLK, HIDDEN), lambda b, s: (b, s, 0)),
        out_shape=jax.ShapeDtypeStruct((B, M, D), jnp.float32),
        scratch_shapes=[pltpu.VMEM((SEGS_PER_BLK, NKEY, VAUG), jnp.bfloat16)],
        compiler_params=pltpu.CompilerParams(
            dimension_semantics=("arbitrary", "arbitrary")),
    )(x, wqkv_bf, wout_bf, perm, dmap)


# speedup vs baseline: 1.1739x; 1.0546x over previous
import functools

import jax
import jax.numpy as jnp
from jax.experimental import pallas as pl
from jax.experimental.pallas import tpu as pltpu

HIDDEN = 1024
HEADS = 16
DH = 64
SEG = 128
SCALE = DH ** -0.5
NKEY = HEADS * SEG
SEGS_PER_BLK = 4
BLK = SEG * SEGS_PER_BLK
GHEADS = 4
GROUPS = HEADS // GHEADS
GROWS = GHEADS * SEG
GVAL = GHEADS * DH
GAUG = GVAL + DH

_TN = (((1,), (1,)), ((), ()))


def _fused_kernel(x_ref, wqkv_ref, wout_ref, perm_ref, dmap_ref, out_ref,
                  vaug_ref):
    b = pl.program_id(0)
    s = pl.program_id(1)

    @pl.when((b == 0) & (s == 0))
    def _init():
        rr = jax.lax.broadcasted_iota(jnp.int32, (GROWS, GAUG), 0)
        cc = jax.lax.broadcasted_iota(jnp.int32, (GROWS, GAUG), 1)
        ini = (cc == GVAL + rr // SEG).astype(jnp.bfloat16)
        for g in range(SEGS_PER_BLK):
            for gr in range(GROUPS):
                vaug_ref[g, gr] = ini

    xb = x_ref[0].astype(jnp.bfloat16)
    qkv = jax.lax.dot_general(xb, wqkv_ref[...], _TN,
                              preferred_element_type=jnp.float32)
    q = qkv[:, :HIDDEN].astype(jnp.bfloat16)
    k = qkv[:, HIDDEN:2 * HIDDEN].astype(jnp.bfloat16)
    v = qkv[:, 2 * HIDDEN:].astype(jnp.bfloat16)

    attn_parts = []
    for g in range(SEGS_PER_BLK):
        r0 = g * SEG
        q_flip = jnp.dot(perm_ref[...], q[r0 + 64:r0 + SEG, :],
                         preferred_element_type=jnp.float32
                         ).astype(jnp.bfloat16)
        q2 = jnp.concatenate([q[r0:r0 + 64, :], q_flip], axis=0)

        parts = []
        for h in range(HEADS):
            sl = slice(h * DH, (h + 1) * DH)
            parts.append(jax.lax.dot_general(
                q2[:, sl], k[r0:r0 + SEG, sl], _TN,
                preferred_element_type=jnp.float32))
            vaug_ref[g, h // GHEADS,
                     (h % GHEADS) * SEG:(h % GHEADS + 1) * SEG,
                     (h % GHEADS) * DH:(h % GHEADS + 1) * DH] = \
                v[r0:r0 + SEG, sl]
        S = jnp.concatenate(parts, axis=1)

        c0 = jnp.max(S[:64, :], axis=0, keepdims=True)
        c1 = jnp.max(S[64:, :], axis=0, keepdims=True)
        W = jnp.exp(S - jnp.concatenate([jnp.broadcast_to(c0, (64, NKEY)),
                                         jnp.broadcast_to(c1, (64, NKEY))],
                                        axis=0))
        Wb = W.astype(jnp.bfloat16)

        seg_parts = []
        for gr in range(GROUPS):
            nd = jnp.dot(Wb[:, gr * GROWS:(gr + 1) * GROWS], vaug_ref[g, gr],
                         preferred_element_type=jnp.float32)
            rec = (1.0 / (1e-10 + nd[:, GVAL:])
                   ).astype(jnp.bfloat16)
            rec_bc = jnp.dot(rec, dmap_ref[...],
                             preferred_element_type=jnp.float32)
            seg_parts.append((nd[:, :GVAL] * rec_bc).astype(jnp.bfloat16))
        attn_parts.append(jnp.concatenate(seg_parts, axis=1))

    attn = jnp.concatenate(attn_parts, axis=0)
    out_ref[0] = jax.lax.dot_general(attn, wout_ref[...], _TN,
                                     preferred_element_type=jnp.float32)


@functools.partial(jax.jit, static_argnums=())
def kernel(x, Wqkv, Wout):
    B, M, D = x.shape
    nblk = M // BLK
    scale_col = jnp.where(jnp.arange(3 * HIDDEN) < HIDDEN, SCALE, 1.0)
    wqkv_bf = (Wqkv * scale_col[:, None]).astype(jnp.bfloat16)
    wout_bf = Wout.astype(jnp.bfloat16)
    i64 = jnp.arange(64)
    perm = (i64[:, None] + i64[None, :] == 63).astype(jnp.bfloat16)
    dmap = (i64[:, None] == jnp.arange(GVAL)[None, :] // DH
            ).astype(jnp.bfloat16)

    return pl.pallas_call(
        _fused_kernel,
        grid=(B, nblk),
        in_specs=[
            pl.BlockSpec((1, BLK, HIDDEN), lambda b, s: (b, s, 0)),
            pl.BlockSpec((3 * HIDDEN, HIDDEN), lambda b, s: (0, 0)),
            pl.BlockSpec((HIDDEN, HIDDEN), lambda b, s: (0, 0)),
            pl.BlockSpec((64, 64), lambda b, s: (0, 0)),
            pl.BlockSpec((64, GVAL), lambda b, s: (0, 0)),
        ],
        out_specs=pl.BlockSpec((1, BLK, HIDDEN), lambda b, s: (b, s, 0)),
        out_shape=jax.ShapeDtypeStruct((B, M, D), jnp.float32),
        scratch_shapes=[pltpu.VMEM((SEGS_PER_BLK, GROUPS, GROWS, GAUG),
                                   jnp.bfloat16)],
        compiler_params=pltpu.CompilerParams(
            dimension_semantics=("arbitrary", "arbitrary")),
    )(x, wqkv_bf, wout_bf, perm, dmap)


# speedup vs baseline: 1.1975x; 1.0201x over previous
import functools

import jax
import jax.numpy as jnp
from jax.experimental import pallas as pl
from jax.experimental.pallas import tpu as pltpu

HIDDEN = 1024
HEADS = 16
DH = 64
SEG = 128
SCALE = DH ** -0.5
NKEY = HEADS * SEG
SEGS_PER_BLK = 8
BLK = SEG * SEGS_PER_BLK
GHEADS = 4
GROUPS = HEADS // GHEADS
GROWS = GHEADS * SEG
GVAL = GHEADS * DH
GAUG = GVAL + DH

_TN = (((1,), (1,)), ((), ()))


def _fused_kernel(x_ref, wqkv_ref, wout_ref, perm_ref, dmap_ref, out_ref,
                  vaug_ref):
    b = pl.program_id(0)
    s = pl.program_id(1)

    @pl.when((b == 0) & (s == 0))
    def _init():
        rr = jax.lax.broadcasted_iota(jnp.int32, (GROWS, GAUG), 0)
        cc = jax.lax.broadcasted_iota(jnp.int32, (GROWS, GAUG), 1)
        ini = (cc == GVAL + rr // SEG).astype(jnp.bfloat16)
        for g in range(SEGS_PER_BLK):
            for gr in range(GROUPS):
                vaug_ref[g, gr] = ini

    xb = x_ref[0].astype(jnp.bfloat16)
    qkv = jax.lax.dot_general(xb, wqkv_ref[...], _TN,
                              preferred_element_type=jnp.float32)
    q = qkv[:, :HIDDEN].astype(jnp.bfloat16)
    k = qkv[:, HIDDEN:2 * HIDDEN].astype(jnp.bfloat16)
    v = qkv[:, 2 * HIDDEN:].astype(jnp.bfloat16)

    attn_parts = []
    for g in range(SEGS_PER_BLK):
        r0 = g * SEG
        q_flip = jnp.dot(perm_ref[...], q[r0 + 64:r0 + SEG, :],
                         preferred_element_type=jnp.float32
                         ).astype(jnp.bfloat16)
        q2 = jnp.concatenate([q[r0:r0 + 64, :], q_flip], axis=0)

        parts = []
        for h in range(HEADS):
            sl = slice(h * DH, (h + 1) * DH)
            parts.append(jax.lax.dot_general(
                q2[:, sl], k[r0:r0 + SEG, sl], _TN,
                preferred_element_type=jnp.float32))
            vaug_ref[g, h // GHEADS,
                     (h % GHEADS) * SEG:(h % GHEADS + 1) * SEG,
                     (h % GHEADS) * DH:(h % GHEADS + 1) * DH] = \
                v[r0:r0 + SEG, sl]
        S = jnp.concatenate(parts, axis=1)

        c0 = jnp.max(S[:64, :], axis=0, keepdims=True)
        c1 = jnp.max(S[64:, :], axis=0, keepdims=True)
        W = jnp.exp(S - jnp.concatenate([jnp.broadcast_to(c0, (64, NKEY)),
                                         jnp.broadcast_to(c1, (64, NKEY))],
                                        axis=0))
        Wb = W.astype(jnp.bfloat16)

        seg_parts = []
        for gr in range(GROUPS):
            nd = jnp.dot(Wb[:, gr * GROWS:(gr + 1) * GROWS], vaug_ref[g, gr],
                         preferred_element_type=jnp.float32)
            rec = (1.0 / (1e-10 + nd[:, GVAL:])
                   ).astype(jnp.bfloat16)
            rec_bc = jnp.dot(rec, dmap_ref[...],
                             preferred_element_type=jnp.float32)
            seg_parts.append((nd[:, :GVAL] * rec_bc).astype(jnp.bfloat16))
        attn_parts.append(jnp.concatenate(seg_parts, axis=1))

    attn = jnp.concatenate(attn_parts, axis=0)
    out_ref[0] = jax.lax.dot_general(attn, wout_ref[...], _TN,
                                     preferred_element_type=jnp.float32)


@functools.partial(jax.jit, static_argnums=())
def kernel(x, Wqkv, Wout):
    B, M, D = x.shape
    nblk = M // BLK
    scale_col = jnp.where(jnp.arange(3 * HIDDEN) < HIDDEN, SCALE, 1.0)
    wqkv_bf = (Wqkv * scale_col[:, None]).astype(jnp.bfloat16)
    wout_bf = Wout.astype(jnp.bfloat16)
    i64 = jnp.arange(64)
    perm = (i64[:, None] + i64[None, :] == 63).astype(jnp.bfloat16)
    dmap = (i64[:, None] == jnp.arange(GVAL)[None, :] // DH
            ).astype(jnp.bfloat16)

    return pl.pallas_call(
        _fused_kernel,
        grid=(B, nblk),
        in_specs=[
            pl.BlockSpec((1, BLK, HIDDEN), lambda b, s: (b, s, 0)),
            pl.BlockSpec((3 * HIDDEN, HIDDEN), lambda b, s: (0, 0)),
            pl.BlockSpec((HIDDEN, HIDDEN), lambda b, s: (0, 0)),
            pl.BlockSpec((64, 64), lambda b, s: (0, 0)),
            pl.BlockSpec((64, GVAL), lambda b, s: (0, 0)),
        ],
        out_specs=pl.BlockSpec((1, BLK, HIDDEN), lambda b, s: (b, s, 0)),
        out_shape=jax.ShapeDtypeStruct((B, M, D), jnp.float32),
        scratch_shapes=[pltpu.VMEM((SEGS_PER_BLK, GROUPS, GROWS, GAUG),
                                   jnp.bfloat16)],
        compiler_params=pltpu.CompilerParams(
            dimension_semantics=("arbitrary", "arbitrary")),
    )(x, wqkv_bf, wout_bf, perm, dmap)
